# baseline (speedup 1.0000x reference)
import functools

import jax
import jax.numpy as jnp
from jax import lax
from jax.experimental import pallas as pl
from jax.experimental.pallas import tpu as pltpu
from jax.experimental.pallas import tpu_sc as plsc

N = 100000
E = 1600000
G = 256
HID = 32
NEXP = 8

NPAD = 100352
PR = NPAD // 8
PBLK = 784
PNB = PR // PBLK
CHK = NPAD // 16

EPAD = 1605632
EROWS = EPAD // 128

RPT = EROWS // 16
ECH = 8
ENOUT = RPT // ECH

RPW = EROWS // 32
DCH = 8
DNOUT = RPW // DCH

QBLK = 128
QNB = PR // QBLK

_MESH = plsc.VectorSubcoreMesh(
    core_axis_name="c", subcore_axis_name="s", num_cores=2, num_subcores=16)

_SC_PARAMS = pltpu.CompilerParams(use_tc_tiling_on_sc=False)


@functools.partial(
    pl.kernel,
    out_type=jax.ShapeDtypeStruct((2 * NPAD,), jnp.float32),
    mesh=_MESH,
    scratch_types=[
        pltpu.VMEM((DCH, 128), jnp.int32),
        pltpu.VMEM((128,), jnp.float32),
        pltpu.VMEM_SHARED((NPAD,), jnp.float32),
    ],
    compiler_params=_SC_PARAMS,
)
def _sc_deg(dst_hbm, zeros_hbm, deg_out, idx_v, ones_v, deg_sp):
    c = lax.axis_index("c")
    s = lax.axis_index("s")

    def _ones(k, carry):
        ones_v[pl.ds(k * 16, 16)] = jnp.full((16,), 1.0, jnp.float32)
        return carry

    lax.fori_loop(0, 8, _ones, 0)
    pltpu.sync_copy(zeros_hbm, deg_sp.at[pl.ds(s * CHK, CHK)])
    plsc.subcore_barrier()

    base = (c * 16 + s) * RPW

    def _outer(q, carry):
        pltpu.sync_copy(dst_hbm.at[pl.ds(base + q * DCH, DCH)], idx_v)

        def _inner(j, carry2):
            pltpu.sync_copy(ones_v, deg_sp.at[idx_v.at[j]], add=True)
            return carry2

        return lax.fori_loop(0, DCH, _inner, carry)

    lax.fori_loop(0, DNOUT, _outer, 0)
    plsc.subcore_barrier()
    pltpu.sync_copy(deg_sp.at[pl.ds(s * CHK, CHK)],
                    deg_out.at[pl.ds(c * NPAD + s * CHK, CHK)])


@functools.partial(
    pl.kernel,
    out_type=jax.ShapeDtypeStruct((2 * NPAD, 16), jnp.float32),
    mesh=_MESH,
    scratch_types=[
        pltpu.VMEM((24, 128), jnp.int32),
        pltpu.VMEM((24, 128), jnp.int32),
        pltpu.VMEM((12, 128, 16), jnp.float32),
        pltpu.VMEM_SHARED((NPAD - 16, 16), jnp.float32),
        pltpu.SemaphoreType.DMA,
        pltpu.SemaphoreType.DMA,
        pltpu.SemaphoreType.DMA,
    ],
    compiler_params=_SC_PARAMS,
)
def _sc_edge(xws_hbm, src2_hbm, dst_hbm, acc_out,
             src_v, dst_v, rows_v, acc_sp, gsem, isem, ssem):
    c = lax.axis_index("c")
    s = lax.axis_index("s")

    @pl.when(s < 15)
    def _():
        pltpu.sync_copy(xws_hbm.at[pl.ds(c * NPAD + s * CHK, CHK)],
                        acc_sp.at[pl.ds(s * CHK, CHK)])

    @pl.when(s == 15)
    def _():
        pltpu.sync_copy(xws_hbm.at[pl.ds(c * NPAD + 15 * CHK, CHK - 16)],
                        acc_sp.at[pl.ds(15 * CHK, CHK - 16)])

    plsc.subcore_barrier()

    base = s * RPT

    def _fire4(islot, half, rslot):
        def _f(j, carry):
            ridx = islot * ECH + half * 4 + j
            pltpu.async_copy(xws_hbm.at[src_v.at[ridx]],
                             rows_v.at[rslot * 4 + j], gsem)
            return carry
        lax.fori_loop(0, 4, _f, 0)

    def _drain_g4():
        def _d(j, carry):
            pltpu.make_async_copy(
                xws_hbm.at[src_v.at[j]], rows_v.at[j], gsem).wait()
            return carry
        lax.fori_loop(0, 4, _d, 0)

    def _scat4(islot, half, rslot):
        def _g(j, carry):
            ridx = islot * ECH + half * 4 + j
            pltpu.async_copy(rows_v.at[rslot * 4 + j],
                             acc_sp.at[dst_v.at[ridx]], ssem, add=True)
            return carry
        lax.fori_loop(0, 4, _g, 0)

    def _wait_s4():
        def _d(j, carry):
            pltpu.make_async_copy(
                rows_v.at[j], acc_sp.at[dst_v.at[0]], ssem).wait()
            return carry
        lax.fori_loop(0, 4, _d, 0)

    def _prefetch(k):
        r1 = base + k * ECH
        sl = lax.rem(k, 3) * ECH
        pltpu.async_copy(src2_hbm.at[c, pl.ds(r1, ECH)],
                         src_v.at[pl.ds(sl, ECH)], isem)
        pltpu.async_copy(dst_hbm.at[pl.ds(r1, ECH)],
                         dst_v.at[pl.ds(sl, ECH)], isem)

    def _drain_i2():
        pltpu.make_async_copy(
            src2_hbm.at[c, pl.ds(base, ECH)],
            src_v.at[pl.ds(0, ECH)], isem).wait()
        pltpu.make_async_copy(
            dst_hbm.at[pl.ds(base, ECH)],
            dst_v.at[pl.ds(0, ECH)], isem).wait()

    HTOT = 2 * ENOUT

    pltpu.sync_copy(src2_hbm.at[c, pl.ds(base, ECH)], src_v.at[pl.ds(0, ECH)])
    pltpu.sync_copy(dst_hbm.at[pl.ds(base, ECH)], dst_v.at[pl.ds(0, ECH)])
    _fire4(0, 0, 0)
    _prefetch(1)

    def _body(h, carry):
        q = lax.div(h, 2)
        half = h - 2 * q
        hn = h + 1
        qn = lax.div(hn, 2)
        halfn = hn - 2 * qn

        @pl.when(hn < HTOT)
        def _():
            @pl.when(hn >= 3)
            def _():
                _wait_s4()

            @pl.when(halfn == 0)
            def _():
                _drain_i2()

            _fire4(lax.rem(qn, 3), halfn, lax.rem(hn, 3))

            @pl.when((halfn == 0) & (qn < ENOUT - 1))
            def _():
                _prefetch(qn + 1)

        _drain_g4()
        _scat4(lax.rem(q, 3), half, lax.rem(h, 3))
        return carry

    lax.fori_loop(0, HTOT, _body, 0)
    _wait_s4()
    _wait_s4()
    _wait_s4()

    plsc.subcore_barrier()

    @pl.when(s < 15)
    def _():
        pltpu.sync_copy(acc_sp.at[pl.ds(s * CHK, CHK)],
                        acc_out.at[pl.ds(c * NPAD + s * CHK, CHK)])

    @pl.when(s == 15)
    def _():
        pltpu.sync_copy(acc_sp.at[pl.ds(15 * CHK, CHK - 16)],
                        acc_out.at[pl.ds(c * NPAD + 15 * CHK, CHK - 16)])


def _first_body(x0_ref, dg_ref, w_ref, dinv_ref, xws_ref):
    dv = lax.rsqrt(1.0 + dg_ref[...])
    dinv_ref[...] = dv
    t = x0_ref[...]
    xws_ref[0] = jnp.dot(t, w_ref[0],
                         preferred_element_type=jnp.float32) * dv
    xws_ref[1] = jnp.dot(t, w_ref[1],
                         preferred_element_type=jnp.float32) * dv


_tc_first = pl.pallas_call(
    _first_body,
    grid=(PNB,),
    in_specs=[
        pl.BlockSpec((PBLK, 128), lambda i: (i, 0)),
        pl.BlockSpec((PBLK, 128), lambda i: (i, 0)),
        pl.BlockSpec((2, 128, 128), lambda i: (0, 0, 0)),
    ],
    out_specs=[
        pl.BlockSpec((PBLK, 128), lambda i: (i, 0)),
        pl.BlockSpec((2, PBLK, 128), lambda i: (0, i, 0)),
    ],
    out_shape=[
        jax.ShapeDtypeStruct((PR, 128), jnp.float32),
        jax.ShapeDtypeStruct((2, PR, 128), jnp.float32),
    ],
)


def _mid_body(acc_ref, dinv_ref, w_ref, b_ref, xws_ref):
    dv = dinv_ref[...]
    t0 = jnp.maximum(acc_ref[0] * dv + b_ref[0, 0][None, :], 0.0)
    t1 = jnp.maximum(acc_ref[1] * dv + b_ref[0, 1][None, :], 0.0)
    y0 = (jnp.dot(t0, w_ref[0, 0], preferred_element_type=jnp.float32)
          + jnp.dot(t1, w_ref[1, 0], preferred_element_type=jnp.float32))
    y1 = (jnp.dot(t0, w_ref[0, 1], preferred_element_type=jnp.float32)
          + jnp.dot(t1, w_ref[1, 1], preferred_element_type=jnp.float32))
    xws_ref[0] = y0 * dv
    xws_ref[1] = y1 * dv


_tc_mid = pl.pallas_call(
    _mid_body,
    grid=(PNB,),
    in_specs=[
        pl.BlockSpec((2, PBLK, 128), lambda i: (0, i, 0)),
        pl.BlockSpec((PBLK, 128), lambda i: (i, 0)),
        pl.BlockSpec((2, 2, 128, 128), lambda i: (0, 0, 0, 0)),
        pl.BlockSpec((1, 2, 128), lambda i: (0, 0, 0)),
    ],
    out_specs=pl.BlockSpec((2, PBLK, 128), lambda i: (0, i, 0)),
    out_shape=jax.ShapeDtypeStruct((2, PR, 128), jnp.float32),
)


GP = G + 8


def _x3_body(acc_ref, dinv_ref, b_ref, x3_ref):
    dv = dinv_ref[...]
    x3_ref[0] = jnp.maximum(acc_ref[0] * dv + b_ref[0, 0][None, :], 0.0)
    x3_ref[1] = jnp.maximum(acc_ref[1] * dv + b_ref[0, 1][None, :], 0.0)


_tc_x3 = pl.pallas_call(
    _x3_body,
    grid=(PNB,),
    in_specs=[
        pl.BlockSpec((2, PBLK, 128), lambda i: (0, i, 0)),
        pl.BlockSpec((PBLK, 128), lambda i: (i, 0)),
        pl.BlockSpec((1, 2, 128), lambda i: (0, 0, 0)),
    ],
    out_specs=pl.BlockSpec((2, PBLK, 128), lambda i: (0, i, 0)),
    out_shape=jax.ShapeDtypeStruct((2, PR, 128), jnp.float32),
)


@functools.partial(
    pl.kernel,
    out_type=[
        jax.ShapeDtypeStruct((2 * GP, 16), jnp.float32),
        jax.ShapeDtypeStruct((2 * GP,), jnp.float32),
    ],
    mesh=_MESH,
    scratch_types=[
        pltpu.VMEM((8, 128), jnp.int32),
        pltpu.VMEM((1024, 16), jnp.float32),
        pltpu.VMEM((128,), jnp.float32),
        pltpu.VMEM_SHARED((GP, 16), jnp.float32),
        pltpu.VMEM_SHARED((GP,), jnp.float32),
    ],
    compiler_params=_SC_PARAMS,
)
def _sc_pool(x3_hbm, batch_hbm, zeros_gp, zeros_g, sums_out, cnt_out,
             bt_v, row_v, ones_v, pool_sp, cnt_sp):
    c = lax.axis_index("c")
    s = lax.axis_index("s")

    def _ones(k, carry):
        ones_v[pl.ds(k * 16, 16)] = jnp.full((16,), 1.0, jnp.float32)
        return carry

    lax.fori_loop(0, 8, _ones, 0)

    @pl.when(s == 0)
    def _():
        pltpu.sync_copy(zeros_gp, pool_sp)
        pltpu.sync_copy(zeros_g, cnt_sp)

    plsc.subcore_barrier()

    n_g = jnp.where(s < 14, 6, 7)
    base_g = jnp.where(s < 14, 6 * s, 84 + 7 * (s - 14))

    def _grp(g9, carry):
        g = base_g + g9
        pltpu.sync_copy(batch_hbm.at[pl.ds(g * 8, 8)], bt_v)
        pltpu.sync_copy(x3_hbm.at[pl.ds(c * NPAD + g * 1024, 1024)], row_v)

        def _sub(j, carry2):
            pltpu.sync_copy(row_v.at[pl.ds(j * 128, 128)],
                            pool_sp.at[bt_v.at[j]], add=True)
            pltpu.sync_copy(ones_v, cnt_sp.at[bt_v.at[j]], add=True)
            return carry2

        return lax.fori_loop(0, 8, _sub, carry)

    lax.fori_loop(0, n_g, _grp, 0)
    plsc.subcore_barrier()

    @pl.when(s == 0)
    def _():
        pltpu.sync_copy(pool_sp, sums_out.at[pl.ds(c * GP, GP)])
        pltpu.sync_copy(cnt_sp, cnt_out.at[pl.ds(c * GP, GP)])


def _pool_body(acc_ref, dinv_ref, b_ref, batch_ref, o_ref):
    i = pl.program_id(0)
    a = jnp.concatenate([acc_ref[0], acc_ref[1]], axis=1)
    dv = dinv_ref[0, :][:, None]
    x3 = jnp.maximum(a * dv + b_ref[0][None, :], 0.0)
    bt = batch_ref[0, :]
    valid2 = bt[:, None] < G
    x3m = jnp.where(valid2, x3, 0.0)
    ones = jnp.where(valid2, 1.0, 0.0)
    xext = jnp.concatenate(
        [x3m, ones, jnp.zeros((QBLK * 8, 31), jnp.float32)], axis=1)
    onehot = (lax.broadcasted_iota(jnp.int32, (G, QBLK * 8), 0)
              == bt[None, :]).astype(jnp.float32)
    contrib = jnp.dot(onehot, xext, preferred_element_type=jnp.float32)

    @pl.when(i == 0)
    def _():
        o_ref[...] = jnp.zeros_like(o_ref)

    o_ref[...] += contrib


_tc_pool = pl.pallas_call(
    _pool_body,
    grid=(QNB,),
    in_specs=[
        pl.BlockSpec((2, QBLK * 8, 16), lambda i: (0, i, 0)),
        pl.BlockSpec((1, QBLK * 8), lambda i: (0, i)),
        pl.BlockSpec((1, HID), lambda i: (0, 0)),
        pl.BlockSpec((1, QBLK * 8), lambda i: (0, i)),
    ],
    out_specs=pl.BlockSpec((G, 64), lambda i: (0, 0)),
    out_shape=jax.ShapeDtypeStruct((G, 64), jnp.float32),
)


def _final_body(p_ref, c_ref, wl_ref, bl_ref, o_ref):
    sums = jnp.concatenate([p_ref[0:G, :], p_ref[GP:GP + G, :]], axis=1)
    cnt = c_ref[0, 0:G][:, None]
    mean = sums / jnp.maximum(cnt, 1.0)
    lg = jnp.dot(mean, wl_ref[...],
                 preferred_element_type=jnp.float32) + bl_ref[0][None, :]
    m = jnp.max(lg, axis=1, keepdims=True)
    e = jnp.exp(lg - m)
    o_ref[...] = e / jnp.sum(e, axis=1, keepdims=True)


_tc_final = pl.pallas_call(
    _final_body,
    out_shape=jax.ShapeDtypeStruct((G, NEXP), jnp.float32),
)


def _expand_w(W):
    eye8 = jnp.eye(8, dtype=jnp.float32)
    blocks = []
    for h in (0, 1):
        row = []
        for hp in (0, 1):
            row.append(jnp.kron(eye8, W[16 * h:16 * h + 16,
                                        16 * hp:16 * hp + 16]))
        blocks.append(jnp.stack(row))
    return jnp.stack(blocks)


@jax.jit
def _run(atomic_numbers, pos, edge_index, batch,
         W1, b1, W2, b2, W3, b3, Wl, bl):
    f32 = jnp.float32
    x0 = jnp.concatenate([atomic_numbers[:, None], pos], axis=1)
    x0p = jnp.pad(x0, ((0, NPAD - N), (0, 12))).reshape(PR, 128)
    er = edge_index.astype(jnp.int32).reshape(2, E // 128, 128).transpose(
        1, 0, 2)
    srcp = jnp.pad(er[:, 0, :], ((0, EROWS - E // 128), (0, 0)))
    dstp = jnp.pad(er[:, 1, :], ((0, EROWS - E // 128), (0, 0)),
                   constant_values=N)
    src2 = (srcp[None, :, :]
            + (jnp.arange(2, dtype=jnp.int32) * NPAD)[:, None, None])
    dst_r = dstp
    batch_p = jnp.pad(batch.astype(jnp.int32), (0, NPAD - N),
                      constant_values=G)[None, :]
    zeros_chk = jnp.zeros((CHK,), f32)

    w1e = _expand_w(jnp.pad(W1, ((0, 28), (0, 0))))[0]
    w2e = _expand_w(W2)
    w3e = _expand_w(W3)
    b1p = jnp.tile(b1.reshape(2, 16), (1, 8)).reshape(1, 2, 128)
    b2p = jnp.tile(b2.reshape(2, 16), (1, 8)).reshape(1, 2, 128)
    b3p = jnp.tile(b3.reshape(2, 16), (1, 8)).reshape(1, 2, 128)

    degs = _sc_deg(dst_r, zeros_chk)
    dg16 = jnp.repeat(degs[:NPAD] + degs[NPAD:], 16).reshape(PR, 128)
    dinv16, xws1 = _tc_first(x0p, dg16, w1e)
    acc1 = _sc_edge(xws1.reshape(2 * NPAD, 16), src2, dst_r)
    xws2 = _tc_mid(acc1.reshape(2, PR, 128), dinv16, w2e, b1p)
    acc2 = _sc_edge(xws2.reshape(2 * NPAD, 16), src2, dst_r)
    xws3 = _tc_mid(acc2.reshape(2, PR, 128), dinv16, w3e, b2p)
    acc3 = _sc_edge(xws3.reshape(2 * NPAD, 16), src2, dst_r)
    x3 = _tc_x3(acc3.reshape(2, PR, 128), dinv16, b3p)
    batch_pr = jnp.pad(batch.astype(jnp.int32), (0, NPAD - N),
                       constant_values=G).reshape(PR // 16, 128)
    sums, cnt = _sc_pool(x3.reshape(2 * NPAD, 16), batch_pr,
                         jnp.zeros((GP, 16), f32), jnp.zeros((GP,), f32))
    return _tc_final(sums, cnt.reshape(1, 2 * GP), Wl, bl.reshape(1, NEXP))


def kernel(atomic_numbers, pos, edge_index, batch,
           W1, b1, W2, b2, W3, b3, Wl, bl):
    return _run(atomic_numbers, pos, edge_index, batch,
                W1, b1, W2, b2, W3, b3, Wl, bl)


# speedup vs baseline: 1.0046x; 1.0046x over previous
import functools

import jax
import jax.numpy as jnp
from jax import lax
from jax.experimental import pallas as pl
from jax.experimental.pallas import tpu as pltpu
from jax.experimental.pallas import tpu_sc as plsc

N = 100000
E = 1600000
G = 256
HID = 32
NEXP = 8

NPAD = 100352
PR = NPAD // 8
PBLK = 784
PNB = PR // PBLK
CHK = NPAD // 16

EPAD = 1605632
EROWS = EPAD // 128

RPT = EROWS // 16
ECH = 8
ENOUT = RPT // ECH

RPW = EROWS // 32
DCH = 8
DNOUT = RPW // DCH

QBLK = 128
QNB = PR // QBLK

_MESH = plsc.VectorSubcoreMesh(
    core_axis_name="c", subcore_axis_name="s", num_cores=2, num_subcores=16)

_SC_PARAMS = pltpu.CompilerParams(use_tc_tiling_on_sc=False)


@functools.partial(
    pl.kernel,
    out_type=jax.ShapeDtypeStruct((2 * NPAD,), jnp.float32),
    mesh=_MESH,
    scratch_types=[
        pltpu.VMEM((DCH, 128), jnp.int32),
        pltpu.VMEM((128,), jnp.float32),
        pltpu.VMEM_SHARED((NPAD,), jnp.float32),
    ],
    compiler_params=_SC_PARAMS,
)
def _sc_deg(dst_hbm, zeros_hbm, deg_out, idx_v, ones_v, deg_sp):
    c = lax.axis_index("c")
    s = lax.axis_index("s")

    def _ones(k, carry):
        ones_v[pl.ds(k * 16, 16)] = jnp.full((16,), 1.0, jnp.float32)
        return carry

    lax.fori_loop(0, 8, _ones, 0)
    pltpu.sync_copy(zeros_hbm, deg_sp.at[pl.ds(s * CHK, CHK)])
    plsc.subcore_barrier()

    base = (c * 16 + s) * RPW

    def _outer(q, carry):
        pltpu.sync_copy(dst_hbm.at[pl.ds(base + q * DCH, DCH)], idx_v)

        def _inner(j, carry2):
            pltpu.sync_copy(ones_v, deg_sp.at[idx_v.at[j]], add=True)
            return carry2

        return lax.fori_loop(0, DCH, _inner, carry)

    lax.fori_loop(0, DNOUT, _outer, 0)
    plsc.subcore_barrier()
    pltpu.sync_copy(deg_sp.at[pl.ds(s * CHK, CHK)],
                    deg_out.at[pl.ds(c * NPAD + s * CHK, CHK)])


@functools.partial(
    pl.kernel,
    out_type=jax.ShapeDtypeStruct((2 * NPAD, 16), jnp.float32),
    mesh=_MESH,
    scratch_types=[
        pltpu.VMEM((24, 128), jnp.int32),
        pltpu.VMEM((24, 128), jnp.int32),
        pltpu.VMEM((8, 128, 16), jnp.float32),
        pltpu.VMEM_SHARED((NPAD, 16), jnp.float32),
        pltpu.SemaphoreType.DMA,
        pltpu.SemaphoreType.DMA,
        pltpu.SemaphoreType.DMA,
    ],
    compiler_params=_SC_PARAMS,
)
def _sc_edge(xws_hbm, src2_hbm, dst_hbm, acc_out,
             src_v, dst_v, rows_v, acc_sp, gsem, isem, ssem):
    c = lax.axis_index("c")
    s = lax.axis_index("s")

    pltpu.sync_copy(xws_hbm.at[pl.ds(c * NPAD + s * CHK, CHK)],
                    acc_sp.at[pl.ds(s * CHK, CHK)])
    plsc.subcore_barrier()

    base = s * RPT

    def _fire4(islot, half, rslot):
        def _f(j, carry):
            ridx = islot * ECH + half * 4 + j
            pltpu.async_copy(xws_hbm.at[src_v.at[ridx]],
                             rows_v.at[rslot * 4 + j], gsem)
            return carry
        lax.fori_loop(0, 4, _f, 0)

    def _drain_g4():
        def _d(j, carry):
            pltpu.make_async_copy(
                xws_hbm.at[src_v.at[j]], rows_v.at[j], gsem).wait()
            return carry
        lax.fori_loop(0, 4, _d, 0)

    def _scat4(islot, half, rslot):
        def _g(j, carry):
            ridx = islot * ECH + half * 4 + j
            pltpu.async_copy(rows_v.at[rslot * 4 + j],
                             acc_sp.at[dst_v.at[ridx]], ssem, add=True)
            return carry
        lax.fori_loop(0, 4, _g, 0)

    def _wait_s4():
        def _d(j, carry):
            pltpu.make_async_copy(
                rows_v.at[j], acc_sp.at[dst_v.at[0]], ssem).wait()
            return carry
        lax.fori_loop(0, 4, _d, 0)

    pltpu.sync_copy(src2_hbm.at[c, pl.ds(base, ECH)], src_v.at[pl.ds(0, ECH)])
    pltpu.sync_copy(dst_hbm.at[pl.ds(base, ECH)], dst_v.at[pl.ds(0, ECH)])

    def _outer(q, carry):
        i_cur = lax.rem(q, 3)
        i_prv = lax.rem(q + 2, 3)
        i_nxt = lax.rem(q + 1, 3)

        @pl.when(q > 0)
        def _():
            _wait_s4()
        _fire4(i_cur, 0, 0)

        @pl.when(q > 0)
        def _():
            _drain_g4()
            _scat4(i_prv, 1, 1)

        @pl.when(q < ENOUT - 1)
        def _():
            r1 = base + (q + 1) * ECH
            pltpu.async_copy(src2_hbm.at[c, pl.ds(r1, ECH)],
                             src_v.at[pl.ds(i_nxt * ECH, ECH)], isem)
            pltpu.async_copy(dst_hbm.at[pl.ds(r1, ECH)],
                             dst_v.at[pl.ds(i_nxt * ECH, ECH)], isem)

        @pl.when(q > 0)
        def _():
            _wait_s4()
        _fire4(i_cur, 1, 1)
        _drain_g4()
        _scat4(i_cur, 0, 0)

        @pl.when(q < ENOUT - 1)
        def _():
            pltpu.make_async_copy(
                src2_hbm.at[c, pl.ds(base, ECH)],
                src_v.at[pl.ds(0, ECH)], isem).wait()
            pltpu.make_async_copy(
                dst_hbm.at[pl.ds(base, ECH)],
                dst_v.at[pl.ds(0, ECH)], isem).wait()

        return carry

    lax.fori_loop(0, ENOUT, _outer, 0)
    _drain_g4()
    _scat4((ENOUT - 1) % 3, 1, 1)
    _wait_s4()
    _wait_s4()

    plsc.subcore_barrier()
    pltpu.sync_copy(acc_sp.at[pl.ds(s * CHK, CHK)],
                    acc_out.at[pl.ds(c * NPAD + s * CHK, CHK)])


def _first_body(x0_ref, dg_ref, w_ref, dinv_ref, xws_ref):
    dv = lax.rsqrt(1.0 + dg_ref[...])
    dinv_ref[...] = dv
    t = x0_ref[...]
    xws_ref[0] = jnp.dot(t, w_ref[0],
                         preferred_element_type=jnp.float32) * dv
    xws_ref[1] = jnp.dot(t, w_ref[1],
                         preferred_element_type=jnp.float32) * dv


_tc_first = pl.pallas_call(
    _first_body,
    grid=(PNB,),
    in_specs=[
        pl.BlockSpec((PBLK, 128), lambda i: (i, 0)),
        pl.BlockSpec((PBLK, 128), lambda i: (i, 0)),
        pl.BlockSpec((2, 128, 128), lambda i: (0, 0, 0)),
    ],
    out_specs=[
        pl.BlockSpec((PBLK, 128), lambda i: (i, 0)),
        pl.BlockSpec((2, PBLK, 128), lambda i: (0, i, 0)),
    ],
    out_shape=[
        jax.ShapeDtypeStruct((PR, 128), jnp.float32),
        jax.ShapeDtypeStruct((2, PR, 128), jnp.float32),
    ],
)


def _mid_body(acc_ref, dinv_ref, w_ref, b_ref, xws_ref):
    dv = dinv_ref[...]
    t0 = jnp.maximum(acc_ref[0] * dv + b_ref[0, 0][None, :], 0.0)
    t1 = jnp.maximum(acc_ref[1] * dv + b_ref[0, 1][None, :], 0.0)
    y0 = (jnp.dot(t0, w_ref[0, 0], preferred_element_type=jnp.float32)
          + jnp.dot(t1, w_ref[1, 0], preferred_element_type=jnp.float32))
    y1 = (jnp.dot(t0, w_ref[0, 1], preferred_element_type=jnp.float32)
          + jnp.dot(t1, w_ref[1, 1], preferred_element_type=jnp.float32))
    xws_ref[0] = y0 * dv
    xws_ref[1] = y1 * dv


_tc_mid = pl.pallas_call(
    _mid_body,
    grid=(PNB,),
    in_specs=[
        pl.BlockSpec((2, PBLK, 128), lambda i: (0, i, 0)),
        pl.BlockSpec((PBLK, 128), lambda i: (i, 0)),
        pl.BlockSpec((2, 2, 128, 128), lambda i: (0, 0, 0, 0)),
        pl.BlockSpec((1, 2, 128), lambda i: (0, 0, 0)),
    ],
    out_specs=pl.BlockSpec((2, PBLK, 128), lambda i: (0, i, 0)),
    out_shape=jax.ShapeDtypeStruct((2, PR, 128), jnp.float32),
)


GP = G + 8


def _x3_body(acc_ref, dinv_ref, b_ref, x3_ref):
    dv = dinv_ref[...]
    x3_ref[0] = jnp.maximum(acc_ref[0] * dv + b_ref[0, 0][None, :], 0.0)
    x3_ref[1] = jnp.maximum(acc_ref[1] * dv + b_ref[0, 1][None, :], 0.0)


_tc_x3 = pl.pallas_call(
    _x3_body,
    grid=(PNB,),
    in_specs=[
        pl.BlockSpec((2, PBLK, 128), lambda i: (0, i, 0)),
        pl.BlockSpec((PBLK, 128), lambda i: (i, 0)),
        pl.BlockSpec((1, 2, 128), lambda i: (0, 0, 0)),
    ],
    out_specs=pl.BlockSpec((2, PBLK, 128), lambda i: (0, i, 0)),
    out_shape=jax.ShapeDtypeStruct((2, PR, 128), jnp.float32),
)


@functools.partial(
    pl.kernel,
    out_type=[
        jax.ShapeDtypeStruct((2 * GP, 16), jnp.float32),
        jax.ShapeDtypeStruct((2 * GP,), jnp.float32),
    ],
    mesh=_MESH,
    scratch_types=[
        pltpu.VMEM((8, 128), jnp.int32),
        pltpu.VMEM((1024, 16), jnp.float32),
        pltpu.VMEM((128,), jnp.float32),
        pltpu.VMEM_SHARED((GP, 16), jnp.float32),
        pltpu.VMEM_SHARED((GP,), jnp.float32),
    ],
    compiler_params=_SC_PARAMS,
)
def _sc_pool(x3_hbm, batch_hbm, zeros_gp, zeros_g, sums_out, cnt_out,
             bt_v, row_v, ones_v, pool_sp, cnt_sp):
    c = lax.axis_index("c")
    s = lax.axis_index("s")

    def _ones(k, carry):
        ones_v[pl.ds(k * 16, 16)] = jnp.full((16,), 1.0, jnp.float32)
        return carry

    lax.fori_loop(0, 8, _ones, 0)

    @pl.when(s == 0)
    def _():
        pltpu.sync_copy(zeros_gp, pool_sp)
        pltpu.sync_copy(zeros_g, cnt_sp)

    plsc.subcore_barrier()

    n_g = jnp.where(s < 14, 6, 7)
    base_g = jnp.where(s < 14, 6 * s, 84 + 7 * (s - 14))

    def _grp(g9, carry):
        g = base_g + g9
        pltpu.sync_copy(batch_hbm.at[pl.ds(g * 8, 8)], bt_v)
        pltpu.sync_copy(x3_hbm.at[pl.ds(c * NPAD + g * 1024, 1024)], row_v)

        def _sub(j, carry2):
            pltpu.sync_copy(row_v.at[pl.ds(j * 128, 128)],
                            pool_sp.at[bt_v.at[j]], add=True)
            pltpu.sync_copy(ones_v, cnt_sp.at[bt_v.at[j]], add=True)
            return carry2

        return lax.fori_loop(0, 8, _sub, carry)

    lax.fori_loop(0, n_g, _grp, 0)
    plsc.subcore_barrier()

    @pl.when(s == 0)
    def _():
        pltpu.sync_copy(pool_sp, sums_out.at[pl.ds(c * GP, GP)])
        pltpu.sync_copy(cnt_sp, cnt_out.at[pl.ds(c * GP, GP)])


def _pool_body(acc_ref, dinv_ref, b_ref, batch_ref, o_ref):
    i = pl.program_id(0)
    a = jnp.concatenate([acc_ref[0], acc_ref[1]], axis=1)
    dv = dinv_ref[0, :][:, None]
    x3 = jnp.maximum(a * dv + b_ref[0][None, :], 0.0)
    bt = batch_ref[0, :]
    valid2 = bt[:, None] < G
    x3m = jnp.where(valid2, x3, 0.0)
    ones = jnp.where(valid2, 1.0, 0.0)
    xext = jnp.concatenate(
        [x3m, ones, jnp.zeros((QBLK * 8, 31), jnp.float32)], axis=1)
    onehot = (lax.broadcasted_iota(jnp.int32, (G, QBLK * 8), 0)
              == bt[None, :]).astype(jnp.float32)
    contrib = jnp.dot(onehot, xext, preferred_element_type=jnp.float32)

    @pl.when(i == 0)
    def _():
        o_ref[...] = jnp.zeros_like(o_ref)

    o_ref[...] += contrib


_tc_pool = pl.pallas_call(
    _pool_body,
    grid=(QNB,),
    in_specs=[
        pl.BlockSpec((2, QBLK * 8, 16), lambda i: (0, i, 0)),
        pl.BlockSpec((1, QBLK * 8), lambda i: (0, i)),
        pl.BlockSpec((1, HID), lambda i: (0, 0)),
        pl.BlockSpec((1, QBLK * 8), lambda i: (0, i)),
    ],
    out_specs=pl.BlockSpec((G, 64), lambda i: (0, 0)),
    out_shape=jax.ShapeDtypeStruct((G, 64), jnp.float32),
)


def _final_body(p_ref, c_ref, wl_ref, bl_ref, o_ref):
    sums = jnp.concatenate([p_ref[0:G, :], p_ref[GP:GP + G, :]], axis=1)
    cnt = c_ref[0, 0:G][:, None]
    mean = sums / jnp.maximum(cnt, 1.0)
    lg = jnp.dot(mean, wl_ref[...],
                 preferred_element_type=jnp.float32) + bl_ref[0][None, :]
    m = jnp.max(lg, axis=1, keepdims=True)
    e = jnp.exp(lg - m)
    o_ref[...] = e / jnp.sum(e, axis=1, keepdims=True)


_tc_final = pl.pallas_call(
    _final_body,
    out_shape=jax.ShapeDtypeStruct((G, NEXP), jnp.float32),
)


def _expand_w(W):
    eye8 = jnp.eye(8, dtype=jnp.float32)
    blocks = []
    for h in (0, 1):
        row = []
        for hp in (0, 1):
            row.append(jnp.kron(eye8, W[16 * h:16 * h + 16,
                                        16 * hp:16 * hp + 16]))
        blocks.append(jnp.stack(row))
    return jnp.stack(blocks)


@jax.jit
def _run(atomic_numbers, pos, edge_index, batch,
         W1, b1, W2, b2, W3, b3, Wl, bl):
    f32 = jnp.float32
    x0 = jnp.concatenate([atomic_numbers[:, None], pos], axis=1)
    x0p = jnp.pad(x0, ((0, NPAD - N), (0, 12))).reshape(PR, 128)
    er = edge_index.astype(jnp.int32).reshape(2, E // 128, 128).transpose(
        1, 0, 2)
    srcp = jnp.pad(er[:, 0, :], ((0, EROWS - E // 128), (0, 0)))
    dstp = jnp.pad(er[:, 1, :], ((0, EROWS - E // 128), (0, 0)),
                   constant_values=N)
    src2 = (srcp[None, :, :]
            + (jnp.arange(2, dtype=jnp.int32) * NPAD)[:, None, None])
    dst_r = dstp
    batch_p = jnp.pad(batch.astype(jnp.int32), (0, NPAD - N),
                      constant_values=G)[None, :]
    zeros_chk = jnp.zeros((CHK,), f32)

    w1e = _expand_w(jnp.pad(W1, ((0, 28), (0, 0))))[0]
    w2e = _expand_w(W2)
    w3e = _expand_w(W3)
    b1p = jnp.tile(b1.reshape(2, 16), (1, 8)).reshape(1, 2, 128)
    b2p = jnp.tile(b2.reshape(2, 16), (1, 8)).reshape(1, 2, 128)
    b3p = jnp.tile(b3.reshape(2, 16), (1, 8)).reshape(1, 2, 128)

    degs = _sc_deg(dst_r, zeros_chk)
    dg16 = jnp.repeat(degs[:NPAD] + degs[NPAD:], 16).reshape(PR, 128)
    dinv16, xws1 = _tc_first(x0p, dg16, w1e)
    acc1 = _sc_edge(xws1.reshape(2 * NPAD, 16), src2, dst_r)
    xws2 = _tc_mid(acc1.reshape(2, PR, 128), dinv16, w2e, b1p)
    acc2 = _sc_edge(xws2.reshape(2 * NPAD, 16), src2, dst_r)
    xws3 = _tc_mid(acc2.reshape(2, PR, 128), dinv16, w3e, b2p)
    acc3 = _sc_edge(xws3.reshape(2 * NPAD, 16), src2, dst_r)
    x3 = _tc_x3(acc3.reshape(2, PR, 128), dinv16, b3p)
    batch_pr = jnp.pad(batch.astype(jnp.int32), (0, NPAD - N),
                       constant_values=G).reshape(PR // 16, 128)
    sums, cnt = _sc_pool(x3.reshape(2 * NPAD, 16), batch_pr,
                         jnp.zeros((GP, 16), f32), jnp.zeros((GP,), f32))
    return _tc_final(sums, cnt.reshape(1, 2 * GP), Wl, bl.reshape(1, NEXP))


def kernel(atomic_numbers, pos, edge_index, batch,
           W1, b1, W2, b2, W3, b3, Wl, bl):
    return _run(atomic_numbers, pos, edge_index, batch,
                W1, b1, W2, b2, W3, b3, Wl, bl)
